# baseline (speedup 1.0000x reference)
import functools

import jax
import jax.numpy as jnp
from jax import lax
from jax.experimental import pallas as pl
from jax.experimental.pallas import tpu as pltpu
from jax.experimental.pallas import tpu_sc as plsc

_NW = 32
_S = 512
_NS = 2
_C = _S * _NS
_BR = 8192


def _gather_body(ids_hbm, tbl_hbm, out_hbm, idx_v, buf_v, sem,
                 *, rows_per_w, n_chunks):
    wid = lax.axis_index("s") * 2 + lax.axis_index("c")
    r_base = wid * rows_per_w

    def chunk_body(ci, _):
        r0 = r_base + ci * _NS
        pltpu.sync_copy(ids_hbm.at[pl.ds(r0, _NS)], idx_v)
        cps = []
        for r in range(_NS):
            cps.append(pltpu.async_copy(
                tbl_hbm.at[idx_v.at[r]], buf_v.at[pl.ds(r * _S, _S)], sem))
        for cp in cps:
            cp.wait()
        pltpu.sync_copy(buf_v, out_hbm.at[pl.ds(r0 * _S, _C)])
        return 0

    lax.fori_loop(0, n_chunks, chunk_body, 0)


def _dequant_body(q_ref, s_ref, o_ref):
    o_ref[...] = q_ref[...].astype(jnp.float32) * s_ref[...]


def kernel(input_ids, q_weight, scale):
    B, L = input_ids.shape
    V, D = q_weight.shape
    N = B * L
    n_rows = N // _S
    rows_per_w = n_rows // _NW
    n_chunks = rows_per_w // _NS
    assert n_rows * _S == N and rows_per_w * _NW == n_rows
    assert n_chunks * _NS == rows_per_w and N % _BR == 0

    ids2 = input_ids.reshape(n_rows, _S).astype(jnp.int32)
    W = D // 4
    qw32 = lax.bitcast_convert_type(q_weight.reshape(V, W, 4), jnp.int32)

    mesh = plsc.VectorSubcoreMesh(core_axis_name="c", subcore_axis_name="s")

    def make_gather(tbl_w, tbl_dtype):
        return pl.kernel(
            functools.partial(_gather_body, rows_per_w=rows_per_w,
                              n_chunks=n_chunks),
            out_type=jax.ShapeDtypeStruct((N, tbl_w), tbl_dtype),
            mesh=mesh,
            scratch_types=[
                pltpu.VMEM((_NS, _S), jnp.int32),
                pltpu.VMEM((_C, tbl_w), tbl_dtype),
                pltpu.SemaphoreType.DMA,
            ],
            compiler_params=pltpu.CompilerParams(
                needs_layout_passes=False, use_tc_tiling_on_sc=False),
        )

    rows32 = make_gather(W, jnp.int32)(ids2, qw32)
    scls = make_gather(1, jnp.float32)(ids2, scale)
    rows = lax.bitcast_convert_type(rows32, jnp.int8).reshape(N, D)

    dequant = pl.pallas_call(
        _dequant_body,
        grid=(N // _BR,),
        in_specs=[
            pl.BlockSpec((_BR, D), lambda i: (i, 0)),
            pl.BlockSpec((_BR, 1), lambda i: (i, 0)),
        ],
        out_specs=pl.BlockSpec((_BR, D), lambda i: (i, 0)),
        out_shape=jax.ShapeDtypeStruct((N, D), jnp.float32),
    )
    return dequant(rows, scls).reshape(B, L, D)


# speedup vs baseline: 1.0061x; 1.0061x over previous
import functools

import jax
import jax.numpy as jnp
from jax import lax
from jax.experimental import pallas as pl
from jax.experimental.pallas import tpu as pltpu
from jax.experimental.pallas import tpu_sc as plsc

_NW = 32
_S = 1600
_NS = 2
_C = _S * _NS
_BR = 8192


def _gather_body(ids_hbm, tbl_hbm, out_hbm, idx_v, buf_v, sem,
                 *, rows_per_w, n_chunks):
    wid = lax.axis_index("s") * 2 + lax.axis_index("c")
    r_base = wid * rows_per_w

    def chunk_body(ci, _):
        r0 = r_base + ci * _NS
        pltpu.sync_copy(ids_hbm.at[pl.ds(r0, _NS)], idx_v)
        cps = []
        for r in range(_NS):
            cps.append(pltpu.async_copy(
                tbl_hbm.at[idx_v.at[r]], buf_v.at[pl.ds(r * _S, _S)], sem))
        for cp in cps:
            cp.wait()
        pltpu.sync_copy(buf_v, out_hbm.at[pl.ds(r0 * _S, _C)])
        return 0

    lax.fori_loop(0, n_chunks, chunk_body, 0)


def _dequant_body(q_ref, s_ref, o_ref):
    o_ref[...] = q_ref[...].astype(jnp.float32) * s_ref[...]


def kernel(input_ids, q_weight, scale):
    B, L = input_ids.shape
    V, D = q_weight.shape
    N = B * L
    n_rows = N // _S
    rows_per_w = n_rows // _NW
    n_chunks = rows_per_w // _NS
    assert n_rows * _S == N and rows_per_w * _NW == n_rows
    assert n_chunks * _NS == rows_per_w and N % _BR == 0

    ids2 = input_ids.reshape(n_rows, _S).astype(jnp.int32)
    W = D // 4
    qw32 = lax.bitcast_convert_type(q_weight.reshape(V, W, 4), jnp.int32)

    mesh = plsc.VectorSubcoreMesh(core_axis_name="c", subcore_axis_name="s")

    def make_gather(tbl_w, tbl_dtype):
        return pl.kernel(
            functools.partial(_gather_body, rows_per_w=rows_per_w,
                              n_chunks=n_chunks),
            out_type=jax.ShapeDtypeStruct((N, tbl_w), tbl_dtype),
            mesh=mesh,
            scratch_types=[
                pltpu.VMEM((_NS, _S), jnp.int32),
                pltpu.VMEM((_C, tbl_w), tbl_dtype),
                pltpu.SemaphoreType.DMA,
            ],
            compiler_params=pltpu.CompilerParams(
                needs_layout_passes=False, use_tc_tiling_on_sc=False),
        )

    rows32 = make_gather(W, jnp.int32)(ids2, qw32)
    scls = make_gather(1, jnp.float32)(ids2, scale)
    rows = lax.bitcast_convert_type(rows32, jnp.int8).reshape(N, D)

    dequant = pl.pallas_call(
        _dequant_body,
        grid=(N // _BR,),
        in_specs=[
            pl.BlockSpec((_BR, D), lambda i: (i, 0)),
            pl.BlockSpec((_BR, 1), lambda i: (i, 0)),
        ],
        out_specs=pl.BlockSpec((_BR, D), lambda i: (i, 0)),
        out_shape=jax.ShapeDtypeStruct((N, D), jnp.float32),
    )
    return dequant(rows, scls).reshape(B, L, D)
